# baseline (speedup 1.0000x reference)
import jax
import jax.numpy as jnp
import numpy as np
from jax import lax
from jax.experimental import pallas as pl
from jax.experimental.pallas import tpu as pltpu
from jax.experimental.pallas import tpu_sc as plsc

_VOCAB = 100000
_B = 1024
_L = 200
_BIAS = float(np.log(12000 / 10000))

_NC = 2
_NS = 16
_NW = _NC * _NS
_ROWS_PER_W = _B // _NW
_IDS_PER_W = _ROWS_PER_W * _L
_HALF = _IDS_PER_W // 2


def _sc_body(idx_hbm, r_hbm, out_hbm, r_sh, idx2_v, idx_v, vals0_v, vals1_v,
             out_v, sem0, sem1):
    sid = lax.axis_index("s")
    wid = sid * _NC + lax.axis_index("c")
    row0 = wid * _ROWS_PER_W
    pltpu.sync_copy(idx_hbm.at[pl.ds(row0, _ROWS_PER_W), :], idx2_v)

    @pl.when(sid == 0)
    def _():
        pltpu.sync_copy(r_hbm, r_sh)

    col_starts = [k * 16 for k in range(12)] + [_L - 16]

    def compact_row(j, _):
        for c in col_starts:
            v = idx2_v[j, pl.ds(c, 16)]
            v = jnp.minimum(jnp.maximum(v, 0), _VOCAB - 1)
            idx_v[pl.ds(j * _L + c, 16)] = v
        return 0

    lax.fori_loop(0, _ROWS_PER_W, compact_row, 0)

    plsc.subcore_barrier()

    cp0 = pltpu.async_copy(r_sh.at[idx_v.at[pl.ds(0, _HALF)]], vals0_v, sem0)
    cp1 = pltpu.async_copy(r_sh.at[idx_v.at[pl.ds(_HALF, _HALF)]], vals1_v, sem1)

    row_off = lax.iota(jnp.int32, 16) * _L

    def make_step(vref):
        def step(l, acc):
            return acc + plsc.load_gather(vref, [row_off + l])
        return step

    init = jnp.full((16,), _BIAS, jnp.float32)
    cp0.wait()
    a0 = lax.fori_loop(0, _L, make_step(vals0_v), init)
    cp1.wait()
    a1 = lax.fori_loop(0, _L, make_step(vals1_v), init)
    out_v[pl.ds(0, 16)] = a0
    out_v[pl.ds(16, 16)] = a1
    pltpu.sync_copy(out_v, out_hbm.at[pl.ds(row0, _ROWS_PER_W)])


@jax.jit
def _run(idx2d, r):
    mesh = plsc.VectorSubcoreMesh(core_axis_name="c", subcore_axis_name="s")
    return pl.kernel(
        _sc_body,
        mesh=mesh,
        compiler_params=pltpu.CompilerParams(needs_layout_passes=False),
        out_type=jax.ShapeDtypeStruct((_B,), jnp.float32),
        scratch_types=[
            pltpu.VMEM_SHARED((_VOCAB,), jnp.float32),
            pltpu.VMEM((_ROWS_PER_W, _L), jnp.int32),
            pltpu.VMEM((_IDS_PER_W,), jnp.int32),
            pltpu.VMEM((_HALF,), jnp.float32),
            pltpu.VMEM((_HALF,), jnp.float32),
            pltpu.VMEM((_ROWS_PER_W,), jnp.float32),
            pltpu.SemaphoreType.DMA,
            pltpu.SemaphoreType.DMA,
        ],
    )(idx2d, r)


def kernel(batch, r):
    return _run(batch.astype(jnp.int32), r)


# speedup vs baseline: 1.1411x; 1.1411x over previous
import jax
import jax.numpy as jnp
import numpy as np
from jax import lax
from jax.experimental import pallas as pl
from jax.experimental.pallas import tpu as pltpu
from jax.experimental.pallas import tpu_sc as plsc

_VOCAB = 100000
_B = 1024
_L = 200
_BIAS = float(np.log(12000 / 10000))

_NC = 2
_NS = 16
_NW = _NC * _NS
_ROWS_PER_W = _B // _NW
_HR = _ROWS_PER_W // 2


def _sc_body(idx_hbm, r_hbm, out_hbm, r_sh, idx_v, vals0_v, vals1_v, out_v,
             sem0, sem1):
    sid = lax.axis_index("s")
    wid = sid * _NC + lax.axis_index("c")
    row0 = wid * _ROWS_PER_W

    @pl.when(sid == 0)
    def _():
        pltpu.sync_copy(r_hbm, r_sh)

    pltpu.sync_copy(idx_hbm.at[pl.ds(row0 * _L, _ROWS_PER_W * _L)], idx_v)

    plsc.subcore_barrier()

    half = _HR * _L
    cp0 = pltpu.async_copy(r_sh.at[idx_v.at[pl.ds(0, half)]], vals0_v, sem0)
    cp1 = pltpu.async_copy(r_sh.at[idx_v.at[pl.ds(half, half)]], vals1_v, sem1)

    row_off = lax.iota(jnp.int32, 16) * _L
    init = (jnp.full((16,), _BIAS, jnp.float32), jnp.zeros((16,), jnp.float32))

    def acc_half(vref):
        def body(l, ab):
            a, b = ab
            p = row_off + 2 * l
            return (a + plsc.load_gather(vref, [p]),
                    b + plsc.load_gather(vref, [p + 1]))
        a, b = plsc.parallel_loop(0, _L // 2, carry=init, unroll=4)(body)
        return a + b

    cp0.wait()
    a0 = acc_half(vals0_v)
    cp1.wait()
    a1 = acc_half(vals1_v)
    out_v[pl.ds(0, 16)] = a0
    out_v[pl.ds(16, 16)] = a1
    pltpu.sync_copy(out_v, out_hbm.at[pl.ds(row0, _ROWS_PER_W)])


@jax.jit
def _run(idx2d, r):
    mesh = plsc.VectorSubcoreMesh(core_axis_name="c", subcore_axis_name="s")
    return pl.kernel(
        _sc_body,
        mesh=mesh,
        compiler_params=pltpu.CompilerParams(
            needs_layout_passes=False, skip_device_barrier=True
        ),
        out_type=jax.ShapeDtypeStruct((_B,), jnp.float32),
        scratch_types=[
            pltpu.VMEM_SHARED((_VOCAB,), jnp.float32),
            pltpu.VMEM((_ROWS_PER_W * _L,), jnp.int32),
            pltpu.VMEM((_HR * _L,), jnp.float32),
            pltpu.VMEM((_HR * _L,), jnp.float32),
            pltpu.VMEM((_ROWS_PER_W,), jnp.float32),
            pltpu.SemaphoreType.DMA,
            pltpu.SemaphoreType.DMA,
        ],
    )(idx2d, r)


def kernel(batch, r):
    return _run(batch.astype(jnp.int32).reshape(-1), r)
